# baseline (speedup 1.0000x reference)
import functools

import jax
import jax.numpy as jnp
from jax import lax
from jax.experimental import pallas as pl
from jax.experimental.pallas import tpu as pltpu
from jax.experimental.pallas import tpu_sc as plsc

_NC, _NS, _L = 2, 16, 16
_NW = _NC * _NS
_B = 4096
_D = 128
_BPW = _B // _NW
_NCH = _D // _L
_MARGIN = 1.0


def _rsqrt(x):
    i = lax.bitcast_convert_type(x, jnp.int32)
    i = jnp.int32(0x5F3759DF) - (i >> 1)
    y = lax.bitcast_convert_type(i, jnp.float32)
    for _ in range(2):
        y = y * (1.5 - 0.5 * x * y * y)
    return y


_ACC = 8
_GRP = _BPW // _L
_GSTRIDE = _ACC * _L * _L


_NSLC = 2
_RPS = _BPW // _NSLC
_GPS = _RPS // _L


def _sc_body(hidx, ridx, tidx, ent, rel, val, cidx, out,
             hidx_v, ridx_v, tidx_v, h_rows, r_rows, t_rows,
             cidx_v, hc_row, out_buf,
             sems, sem_c):
    wid = lax.axis_index("s") * _NC + lax.axis_index("c")
    base = wid * _BPW

    pltpu.sync_copy(hidx.at[pl.ds(base, _BPW)], hidx_v)
    pltpu.sync_copy(ridx.at[pl.ds(base, _BPW)], ridx_v)
    pltpu.sync_copy(tidx.at[pl.ds(base, _BPW)], tidx_v)
    pltpu.sync_copy(cidx, cidx_v)

    cp_h = pltpu.async_copy(ent.at[hidx_v], h_rows, sems.at[0, 0])
    cp_r = pltpu.async_copy(rel.at[ridx_v], r_rows, sems.at[0, 1])
    cp_t = pltpu.async_copy(val.at[tidx_v], t_rows, sems.at[0, 2])
    cp_c = pltpu.async_copy(ent.at[cidx_v], hc_row, sem_c)

    lanes = lax.iota(jnp.int32, _L)

    cp_c.wait()
    cc = jnp.zeros((_L,), jnp.float32)
    chunks = []
    for c in range(_NCH):
        v = hc_row[0, pl.ds(c * _L, _L)]
        chunks.append(v)
        cc = cc + v * v
    inc = _rsqrt(jnp.maximum(jnp.sum(cc), 1e-24))
    hcn = [chunks[c] * inc for c in range(_NCH)]

    def _group_body(g):
        gb = g * _L
        vec = jnp.zeros((_L,), jnp.float32)
        for j in range(_L):
            i = gb + j
            z = jnp.zeros((_L,), jnp.float32)
            hh = tt = z
            hs, rs, ts = [], [], []
            for c in range(_NCH):
                sl = pl.ds(c * _L, _L)
                h = h_rows[i, sl]
                r = r_rows[i, sl]
                t = t_rows[i, sl]
                hs.append(h)
                rs.append(r)
                ts.append(t)
                hh = hh + h * h
                tt = tt + t * t
            inh = _rsqrt(jnp.maximum(jnp.sum(hh), 1e-24))
            int_ = _rsqrt(jnp.maximum(jnp.sum(tt), 1e-24))
            pos_a = neg_a = z
            for c in range(_NCH):
                u = rs[c] - ts[c] * int_
                d = hs[c] * inh + u
                e = hcn[c] + u
                pos_a = pos_a + d * d
                neg_a = neg_a + e * e
            pos2 = jnp.sum(pos_a)
            neg2 = jnp.sum(neg_a)
            pos = pos2 * _rsqrt(jnp.maximum(pos2, 1e-30))
            neg = neg2 * _rsqrt(jnp.maximum(neg2, 1e-30))
            vec = jnp.where(lanes == j, pos - neg + _MARGIN, vec)
        off = pl.multiple_of(g * _L, _L)
        out_buf[pl.ds(off, _L)] = vec

    cp_h.wait()
    cp_r.wait()
    cp_t.wait()

    @plsc.parallel_loop(0, _GRP)
    def _group(g):
        _group_body(g)

    pltpu.sync_copy(out_buf, out.at[pl.ds(base, _BPW)])


_sc_kernel = functools.partial(
    pl.kernel,
    out_type=jax.ShapeDtypeStruct((_B,), jnp.float32),
    mesh=plsc.VectorSubcoreMesh(core_axis_name="c", subcore_axis_name="s"),
    compiler_params=pltpu.CompilerParams(needs_layout_passes=False),
    scratch_types=[
        pltpu.VMEM((_BPW,), jnp.int32),
        pltpu.VMEM((_BPW,), jnp.int32),
        pltpu.VMEM((_BPW,), jnp.int32),
        pltpu.VMEM((_BPW, _D), jnp.float32),
        pltpu.VMEM((_BPW, _D), jnp.float32),
        pltpu.VMEM((_BPW, _D), jnp.float32),
        pltpu.VMEM((1,), jnp.int32),
        pltpu.VMEM((1, _D), jnp.float32),
        pltpu.VMEM((_BPW,), jnp.float32),
        pltpu.SemaphoreType.DMA((_NSLC, 3)),
        pltpu.SemaphoreType.DMA,
    ],
)(_sc_body)


def kernel(data, ent_embeds, rel_embeds, val_embeds, corrupt_idx):
    hidx = data[:, 0]
    ridx = data[:, 1]
    tidx = data[:, 2]
    return _sc_kernel(hidx, ridx, tidx,
                      ent_embeds, rel_embeds, val_embeds, corrupt_idx)


# speedup vs baseline: 1.1054x; 1.1054x over previous
import functools

import jax
import jax.numpy as jnp
from jax import lax
from jax.experimental import pallas as pl
from jax.experimental.pallas import tpu as pltpu
from jax.experimental.pallas import tpu_sc as plsc

_NC, _NS, _L = 2, 16, 16
_NW = _NC * _NS
_B = 4096
_D = 128
_BPW = _B // _NW
_NCH = _D // _L
_MARGIN = 1.0


def _rsqrt(x):
    i = lax.bitcast_convert_type(x, jnp.int32)
    i = jnp.int32(0x5F3759DF) - (i >> 1)
    y = lax.bitcast_convert_type(i, jnp.float32)
    for _ in range(2):
        y = y * (1.5 - 0.5 * x * y * y)
    return y


_ACC = 8
_GRP = _BPW // _L
_GSTRIDE = _ACC * _L * _L


_NSLC = 2
_RPS = _BPW // _NSLC
_GPS = _RPS // _L


def _sc_body(hidx, ridx, tidx, ent, rel, val, cidx, out,
             hidx_v, ridx_v, tidx_v, h_rows, r_rows, t_rows,
             cidx_v, hc_row, out_buf,
             sems, sem_c):
    wid = lax.axis_index("s") * _NC + lax.axis_index("c")
    base = wid * _BPW

    pltpu.sync_copy(hidx.at[pl.ds(base, _BPW)], hidx_v)
    pltpu.sync_copy(ridx.at[pl.ds(base, _BPW)], ridx_v)
    pltpu.sync_copy(tidx.at[pl.ds(base, _BPW)], tidx_v)
    pltpu.sync_copy(cidx, cidx_v)

    cp_h = pltpu.async_copy(ent.at[hidx_v], h_rows, sems.at[0, 0])
    cp_r = pltpu.async_copy(rel.at[ridx_v], r_rows, sems.at[0, 1])
    cp_t = pltpu.async_copy(val.at[tidx_v], t_rows, sems.at[0, 2])
    cp_c = pltpu.async_copy(ent.at[cidx_v], hc_row, sem_c)

    lanes = lax.iota(jnp.int32, _L)

    cp_c.wait()
    cc = jnp.zeros((_L,), jnp.float32)
    chunks = []
    for c in range(_NCH):
        v = hc_row[0, pl.ds(c * _L, _L)]
        chunks.append(v)
        cc = cc + v * v
    cc_s = jnp.sum(cc)
    inc = _rsqrt(jnp.maximum(cc_s, 1e-24))
    hcn = [chunks[c] * inc for c in range(_NCH)]
    ccn_sc = cc_s * inc * inc

    def _group_body(g):
        gb = g * _L
        vec = jnp.zeros((_L,), jnp.float32)
        for j in range(_L):
            i = gb + j
            z = jnp.zeros((_L,), jnp.float32)
            hh = tt = rr = hr = ht = rt = cr = ct = z
            for c in range(_NCH):
                sl = pl.ds(c * _L, _L)
                h = h_rows[i, sl]
                r = r_rows[i, sl]
                t = t_rows[i, sl]
                hh = hh + h * h
                tt = tt + t * t
                rr = rr + r * r
                hr = hr + h * r
                ht = ht + h * t
                rt = rt + r * t
                cr = cr + hcn[c] * r
                ct = ct + hcn[c] * t
            hh_s = jnp.sum(hh)
            tt_s = jnp.sum(tt)
            rr_s = jnp.sum(rr)
            hr_s = jnp.sum(hr)
            ht_s = jnp.sum(ht)
            rt_s = jnp.sum(rt)
            cr_s = jnp.sum(cr)
            ct_s = jnp.sum(ct)
            inh = _rsqrt(jnp.maximum(hh_s, 1e-24))
            int_ = _rsqrt(jnp.maximum(tt_s, 1e-24))
            hn2 = hh_s * inh * inh
            tn2 = tt_s * int_ * int_
            pos2 = (hn2 + tn2 + rr_s
                    + 2.0 * hr_s * inh - 2.0 * ht_s * inh * int_
                    - 2.0 * rt_s * int_)
            neg2 = (ccn_sc + tn2 + rr_s
                    + 2.0 * cr_s - 2.0 * ct_s * int_ - 2.0 * rt_s * int_)
            pos2 = jnp.maximum(pos2, 0.0)
            neg2 = jnp.maximum(neg2, 0.0)
            pos = pos2 * _rsqrt(jnp.maximum(pos2, 1e-30))
            neg = neg2 * _rsqrt(jnp.maximum(neg2, 1e-30))
            vec = jnp.where(lanes == j, pos - neg + _MARGIN, vec)
        off = pl.multiple_of(g * _L, _L)
        out_buf[pl.ds(off, _L)] = vec

    cp_h.wait()
    cp_r.wait()
    cp_t.wait()

    @plsc.parallel_loop(0, _GRP)
    def _group(g):
        _group_body(g)

    pltpu.sync_copy(out_buf, out.at[pl.ds(base, _BPW)])


_sc_kernel = functools.partial(
    pl.kernel,
    out_type=jax.ShapeDtypeStruct((_B,), jnp.float32),
    mesh=plsc.VectorSubcoreMesh(core_axis_name="c", subcore_axis_name="s"),
    compiler_params=pltpu.CompilerParams(needs_layout_passes=False),
    scratch_types=[
        pltpu.VMEM((_BPW,), jnp.int32),
        pltpu.VMEM((_BPW,), jnp.int32),
        pltpu.VMEM((_BPW,), jnp.int32),
        pltpu.VMEM((_BPW, _D), jnp.float32),
        pltpu.VMEM((_BPW, _D), jnp.float32),
        pltpu.VMEM((_BPW, _D), jnp.float32),
        pltpu.VMEM((1,), jnp.int32),
        pltpu.VMEM((1, _D), jnp.float32),
        pltpu.VMEM((_BPW,), jnp.float32),
        pltpu.SemaphoreType.DMA((_NSLC, 3)),
        pltpu.SemaphoreType.DMA,
    ],
)(_sc_body)


def kernel(data, ent_embeds, rel_embeds, val_embeds, corrupt_idx):
    hidx = data[:, 0]
    ridx = data[:, 1]
    tidx = data[:, 2]
    return _sc_kernel(hidx, ridx, tidx,
                      ent_embeds, rel_embeds, val_embeds, corrupt_idx)


# speedup vs baseline: 1.1122x; 1.0062x over previous
import functools

import jax
import jax.numpy as jnp
from jax import lax
from jax.experimental import pallas as pl
from jax.experimental.pallas import tpu as pltpu
from jax.experimental.pallas import tpu_sc as plsc

_NC, _NS, _L = 2, 16, 16
_NW = _NC * _NS
_B = 4096
_D = 128
_BPW = _B // _NW
_NCH = _D // _L
_MARGIN = 1.0


def _rsqrt(x):
    i = lax.bitcast_convert_type(x, jnp.int32)
    i = jnp.int32(0x5F3759DF) - (i >> 1)
    y = lax.bitcast_convert_type(i, jnp.float32)
    for _ in range(2):
        y = y * (1.5 - 0.5 * x * y * y)
    return y


_ACC = 8
_GRP = _BPW // _L
_GSTRIDE = _ACC * _L * _L


_NSLC = 2
_RPS = _BPW // _NSLC
_GPS = _RPS // _L


def _sc_body(hidx, ridx, tidx, ent, rel, val, cidx, out,
             hidx_v, ridx_v, tidx_v, h_rows, r_rows, t_rows,
             cidx_v, hc_row, out_buf,
             sems, sem_c):
    wid = lax.axis_index("s") * _NC + lax.axis_index("c")
    base = wid * _BPW

    pltpu.sync_copy(hidx.at[pl.ds(base, _BPW)], hidx_v)
    pltpu.sync_copy(ridx.at[pl.ds(base, _BPW)], ridx_v)
    pltpu.sync_copy(tidx.at[pl.ds(base, _BPW)], tidx_v)
    pltpu.sync_copy(cidx, cidx_v)

    sl0 = pl.ds(0, _RPS)
    sl1 = pl.ds(_RPS, _RPS)
    cp_h0 = pltpu.async_copy(ent.at[hidx_v.at[sl0]], h_rows.at[sl0], sems.at[0, 0])
    cp_r0 = pltpu.async_copy(rel.at[ridx_v.at[sl0]], r_rows.at[sl0], sems.at[0, 1])
    cp_t0 = pltpu.async_copy(val.at[tidx_v.at[sl0]], t_rows.at[sl0], sems.at[0, 2])
    cp_h1 = pltpu.async_copy(ent.at[hidx_v.at[sl1]], h_rows.at[sl1], sems.at[1, 0])
    cp_r1 = pltpu.async_copy(rel.at[ridx_v.at[sl1]], r_rows.at[sl1], sems.at[1, 1])
    cp_t1 = pltpu.async_copy(val.at[tidx_v.at[sl1]], t_rows.at[sl1], sems.at[1, 2])
    cp_c = pltpu.async_copy(ent.at[cidx_v], hc_row, sem_c)

    lanes = lax.iota(jnp.int32, _L)

    cp_c.wait()
    cc = jnp.zeros((_L,), jnp.float32)
    chunks = []
    for c in range(_NCH):
        v = hc_row[0, pl.ds(c * _L, _L)]
        chunks.append(v)
        cc = cc + v * v
    cc_s = jnp.sum(cc)
    inc = _rsqrt(jnp.maximum(cc_s, 1e-24))
    hcn = [chunks[c] * inc for c in range(_NCH)]
    ccn_sc = cc_s * inc * inc

    def _group_body(g):
        gb = g * _L
        vec = jnp.zeros((_L,), jnp.float32)
        for j in range(_L):
            i = gb + j
            z = jnp.zeros((_L,), jnp.float32)
            hh = tt = rr = hr = ht = rt = cr = ct = z
            for c in range(_NCH):
                sl = pl.ds(c * _L, _L)
                h = h_rows[i, sl]
                r = r_rows[i, sl]
                t = t_rows[i, sl]
                hh = hh + h * h
                tt = tt + t * t
                rr = rr + r * r
                hr = hr + h * r
                ht = ht + h * t
                rt = rt + r * t
                cr = cr + hcn[c] * r
                ct = ct + hcn[c] * t
            hh_s = jnp.sum(hh)
            tt_s = jnp.sum(tt)
            rr_s = jnp.sum(rr)
            hr_s = jnp.sum(hr)
            ht_s = jnp.sum(ht)
            rt_s = jnp.sum(rt)
            cr_s = jnp.sum(cr)
            ct_s = jnp.sum(ct)
            inh = _rsqrt(jnp.maximum(hh_s, 1e-24))
            int_ = _rsqrt(jnp.maximum(tt_s, 1e-24))
            hn2 = hh_s * inh * inh
            tn2 = tt_s * int_ * int_
            pos2 = (hn2 + tn2 + rr_s
                    + 2.0 * hr_s * inh - 2.0 * ht_s * inh * int_
                    - 2.0 * rt_s * int_)
            neg2 = (ccn_sc + tn2 + rr_s
                    + 2.0 * cr_s - 2.0 * ct_s * int_ - 2.0 * rt_s * int_)
            pos2 = jnp.maximum(pos2, 0.0)
            neg2 = jnp.maximum(neg2, 0.0)
            pos = pos2 * _rsqrt(jnp.maximum(pos2, 1e-30))
            neg = neg2 * _rsqrt(jnp.maximum(neg2, 1e-30))
            vec = jnp.where(lanes == j, pos - neg + _MARGIN, vec)
        off = pl.multiple_of(g * _L, _L)
        out_buf[pl.ds(off, _L)] = vec

    cp_h0.wait()
    cp_r0.wait()
    cp_t0.wait()

    def _group(g, carry):
        @pl.when(g == _GRP // 2)
        def _():
            cp_h1.wait()
            cp_r1.wait()
            cp_t1.wait()

        _group_body(g)
        return carry

    lax.fori_loop(0, _GRP, _group, 0)

    pltpu.sync_copy(out_buf, out.at[pl.ds(base, _BPW)])


_sc_kernel = functools.partial(
    pl.kernel,
    out_type=jax.ShapeDtypeStruct((_B,), jnp.float32),
    mesh=plsc.VectorSubcoreMesh(core_axis_name="c", subcore_axis_name="s"),
    compiler_params=pltpu.CompilerParams(needs_layout_passes=False),
    scratch_types=[
        pltpu.VMEM((_BPW,), jnp.int32),
        pltpu.VMEM((_BPW,), jnp.int32),
        pltpu.VMEM((_BPW,), jnp.int32),
        pltpu.VMEM((_BPW, _D), jnp.float32),
        pltpu.VMEM((_BPW, _D), jnp.float32),
        pltpu.VMEM((_BPW, _D), jnp.float32),
        pltpu.VMEM((1,), jnp.int32),
        pltpu.VMEM((1, _D), jnp.float32),
        pltpu.VMEM((_BPW,), jnp.float32),
        pltpu.SemaphoreType.DMA((_NSLC, 3)),
        pltpu.SemaphoreType.DMA,
    ],
)(_sc_body)


def kernel(data, ent_embeds, rel_embeds, val_embeds, corrupt_idx):
    hidx = data[:, 0]
    ridx = data[:, 1]
    tidx = data[:, 2]
    return _sc_kernel(hidx, ridx, tidx,
                      ent_embeds, rel_embeds, val_embeds, corrupt_idx)
